# baseline (device time: 1483117 ns/iter reference)
import jax
import jax.numpy as jnp
from jax import lax
from jax.experimental import pallas as pl
from jax.experimental.pallas import tpu as pltpu

N_DEV = 32


def kernel(x, w_mat, scale_x, scale_w):
    m, _ = x.shape
    _, n = w_mat.shape
    mo = m // N_DEV

    def body(x_ref, w_ref, sx_ref, sw_ref, out_ref,
             send_buf, recv_buf, send_sems, recv_sems, credit_sem):
        my = lax.axis_index("i")
        left = lax.rem(my + (N_DEV - 1), N_DEV)
        right = lax.rem(my + 1, N_DEV)

        barrier = pltpu.get_barrier_semaphore()
        pl.semaphore_signal(barrier, inc=1, device_id=(left,),
                            device_id_type=pl.DeviceIdType.MESH)
        pl.semaphore_signal(barrier, inc=1, device_id=(right,),
                            device_id_type=pl.DeviceIdType.MESH)
        pl.semaphore_wait(barrier, 2)

        w_bf = w_ref[...].astype(jnp.bfloat16)

        def partial_block(step):
            c = lax.rem(my + (2 * N_DEV - 1 - step), N_DEV)
            xs = x_ref[pl.ds(c * mo, mo), :].astype(jnp.bfloat16)
            return jax.lax.dot_general(
                xs, w_bf, (((1,), (0,)), ((), ())),
                preferred_element_type=jnp.float32)

        def make_rdma(slot):
            return pltpu.make_async_remote_copy(
                src_ref=send_buf.at[slot],
                dst_ref=recv_buf.at[slot],
                send_sem=send_sems.at[slot],
                recv_sem=recv_sems.at[slot],
                device_id=(right,),
                device_id_type=pl.DeviceIdType.MESH,
            )

        rdmas = {}
        for s in range(N_DEV):
            p = partial_block(s)
            if s == 0:
                acc = p
            else:
                rdmas[s - 1].wait_recv()
                acc = recv_buf[(s - 1) % 2, :, :] + p
                if s <= N_DEV - 3:
                    pl.semaphore_signal(credit_sem, inc=1, device_id=(left,),
                                        device_id_type=pl.DeviceIdType.MESH)
            if s <= N_DEV - 2:
                slot = s % 2
                if s >= 2:
                    rdmas[s - 2].wait_send()
                    pl.semaphore_wait(credit_sem, 1)
                send_buf[slot, :, :] = acc
                rdmas[s] = make_rdma(slot)
                rdmas[s].start()
            else:
                out_ref[...] = acc * (sx_ref[0] * sw_ref[0])

        rdmas[N_DEV - 3].wait_send()
        rdmas[N_DEV - 2].wait_send()

    return pl.pallas_call(
        body,
        out_shape=jax.ShapeDtypeStruct((mo, n), jnp.float32),
        in_specs=[
            pl.BlockSpec(memory_space=pltpu.VMEM),
            pl.BlockSpec(memory_space=pltpu.VMEM),
            pl.BlockSpec(memory_space=pltpu.SMEM),
            pl.BlockSpec(memory_space=pltpu.SMEM),
        ],
        out_specs=pl.BlockSpec(memory_space=pltpu.VMEM),
        scratch_shapes=[
            pltpu.VMEM((2, mo, n), jnp.float32),
            pltpu.VMEM((2, mo, n), jnp.float32),
            pltpu.SemaphoreType.DMA((2,)),
            pltpu.SemaphoreType.DMA((2,)),
            pltpu.SemaphoreType.REGULAR,
        ],
        compiler_params=pltpu.CompilerParams(collective_id=0),
    )(x, w_mat, scale_x, scale_w)


# device time: 785000 ns/iter; 1.8893x vs baseline; 1.8893x over previous
import jax
import jax.numpy as jnp
from jax import lax
from jax.experimental import pallas as pl
from jax.experimental.pallas import tpu as pltpu

N_DEV = 32

_RING = [0, 3, 4, 7, 15, 12, 11, 8, 16, 19, 20, 23, 31, 28, 27, 24,
         25, 26, 29, 30, 22, 21, 18, 17, 9, 10, 13, 14, 6, 5, 2, 1]


def kernel(x, w_mat, scale_x, scale_w):
    m, _ = x.shape
    _, n = w_mat.shape
    mo = m // N_DEV
    nh = n // 2

    ring = jnp.asarray(_RING, jnp.int32)
    my = lax.axis_index("i")
    v = jnp.argmax(ring == my).astype(jnp.int32)
    steps = jnp.arange(N_DEV, dtype=jnp.int32)
    cAs = ring[(v - 1 - steps) % N_DEV]
    cBs = ring[(v + 1 + steps) % N_DEV]
    nbrs = jnp.stack([ring[(v + 1) % N_DEV], ring[(v - 1) % N_DEV]])

    def body(cAs_ref, cBs_ref, nbr_ref, x_ref, w_ref, sx_ref, sw_ref, out_ref,
             sbA, rbA, sbB, rbB, ssA, rsA, ssB, rsB, credA, credB):
        nxt = nbr_ref[0]
        prv = nbr_ref[1]

        barrier = pltpu.get_barrier_semaphore()
        for nb in (nxt, prv):
            pl.semaphore_signal(barrier, inc=1, device_id=(nb,),
                                device_id_type=pl.DeviceIdType.MESH)
        pl.semaphore_wait(barrier, 2)

        w_bf = w_ref[...].astype(jnp.bfloat16)

        def pblock(c, lo):
            xs = x_ref[pl.ds(c * mo, mo), :].astype(jnp.bfloat16)
            return jax.lax.dot_general(
                xs, w_bf[:, lo:lo + nh], (((1,), (0,)), ((), ())),
                preferred_element_type=jnp.float32)

        def mk(sbuf, rbuf, ssem, rsem, slot, dev):
            return pltpu.make_async_remote_copy(
                src_ref=sbuf.at[slot], dst_ref=rbuf.at[slot],
                send_sem=ssem.at[slot], recv_sem=rsem.at[slot],
                device_id=(dev,), device_id_type=pl.DeviceIdType.MESH)

        rdA, rdB = {}, {}
        for s in range(N_DEV):
            pA = pblock(cAs_ref[s], 0)
            pB = pblock(cBs_ref[s], nh)
            slot = s % 2
            if s == 0:
                accA, accB = pA, pB
            else:
                rslot = (s - 1) % 2
                rdA[s - 1].wait_recv()
                accA = rbA[rslot, :, :] + pA
                rdB[s - 1].wait_recv()
                accB = rbB[rslot, :, :] + pB
                if s <= N_DEV - 3:
                    pl.semaphore_signal(credA, inc=1, device_id=(prv,),
                                        device_id_type=pl.DeviceIdType.MESH)
                    pl.semaphore_signal(credB, inc=1, device_id=(nxt,),
                                        device_id_type=pl.DeviceIdType.MESH)
            if s <= N_DEV - 2:
                if s >= 2:
                    rdA[s - 2].wait_send()
                    pl.semaphore_wait(credA, 1)
                    rdB[s - 2].wait_send()
                    pl.semaphore_wait(credB, 1)
                sbA[slot, :, :] = accA
                rdA[s] = mk(sbA, rbA, ssA, rsA, slot, nxt)
                rdA[s].start()
                sbB[slot, :, :] = accB
                rdB[s] = mk(sbB, rbB, ssB, rsB, slot, prv)
                rdB[s].start()
            else:
                sc = sx_ref[0] * sw_ref[0]
                out_ref[:, :nh] = accA * sc
                out_ref[:, nh:] = accB * sc

        for rd in (rdA, rdB):
            rd[N_DEV - 3].wait_send()
            rd[N_DEV - 2].wait_send()

    return pl.pallas_call(
        body,
        out_shape=jax.ShapeDtypeStruct((mo, n), jnp.float32),
        in_specs=[
            pl.BlockSpec(memory_space=pltpu.SMEM),
            pl.BlockSpec(memory_space=pltpu.SMEM),
            pl.BlockSpec(memory_space=pltpu.SMEM),
            pl.BlockSpec(memory_space=pltpu.VMEM),
            pl.BlockSpec(memory_space=pltpu.VMEM),
            pl.BlockSpec(memory_space=pltpu.SMEM),
            pl.BlockSpec(memory_space=pltpu.SMEM),
        ],
        out_specs=pl.BlockSpec(memory_space=pltpu.VMEM),
        scratch_shapes=[
            pltpu.VMEM((2, mo, nh), jnp.float32),
            pltpu.VMEM((2, mo, nh), jnp.float32),
            pltpu.VMEM((2, mo, nh), jnp.float32),
            pltpu.VMEM((2, mo, nh), jnp.float32),
            pltpu.SemaphoreType.DMA((2,)),
            pltpu.SemaphoreType.DMA((2,)),
            pltpu.SemaphoreType.DMA((2,)),
            pltpu.SemaphoreType.DMA((2,)),
            pltpu.SemaphoreType.REGULAR,
            pltpu.SemaphoreType.REGULAR,
        ],
        compiler_params=pltpu.CompilerParams(collective_id=0),
    )(cAs, cBs, nbrs, x, w_mat, scale_x, scale_w)
